# baseline (speedup 1.0000x reference)
import functools

import jax
import jax.numpy as jnp
from jax import lax
from jax.experimental import pallas as pl
from jax.experimental.pallas import tpu as pltpu
from jax.experimental.pallas import tpu_sc as plsc

_LANES = 16
_ROWS_PER_CHUNK = 8
_UNROLL = 8
_NRING = 3


def _make_sc_kernel(batch, seq_len, d_model):
    n_workers = 32
    rows_per_w = seq_len // n_workers
    chunk = _ROWS_PER_CHUNK
    n_chunks = rows_per_w // chunk
    vregs_per_row = d_model // _LANES

    mesh = plsc.VectorSubcoreMesh(core_axis_name="c", subcore_axis_name="s")

    xbuf_types = [
        pltpu.VMEM((chunk, d_model), jnp.float32)
        for _ in range(_NRING * batch)
    ]
    tbuf_types = [pltpu.VMEM((chunk, d_model), jnp.float32) for _ in range(2)]

    @functools.partial(
        pl.kernel,
        mesh=mesh,
        out_type=jax.ShapeDtypeStruct((batch, seq_len, d_model), jnp.float32),
        scratch_types=tbuf_types + xbuf_types + [
            pltpu.SemaphoreType.DMA,
            pltpu.SemaphoreType.DMA,
            pltpu.SemaphoreType.DMA,
        ],
    )
    def sc_kernel(x_hbm, tab_hbm, out_hbm, *refs):
        tbufs = list(refs[:2])
        xbufs = [list(refs[2 + p * batch: 2 + (p + 1) * batch])
                 for p in range(_NRING)]
        st, sx, so = refs[2 + _NRING * batch:]

        wid = lax.axis_index("s") * 2 + lax.axis_index("c")
        row_base = wid * rows_per_w

        t_cp = [None, None]
        x_cp = [None] * _NRING
        o_cp = [None] * _NRING

        def issue_xin(c):
            p = c % _NRING
            cps = []
            for b in range(batch):
                cps.append(pltpu.async_copy(
                    x_hbm.at[b, pl.ds(row_base + c * chunk, chunk)],
                    xbufs[p][b], sx))
            x_cp[p] = cps

        t_cp[0] = pltpu.async_copy(
            tab_hbm.at[pl.ds(row_base, chunk)], tbufs[0], st)
        for c in range(min(_NRING - 1, n_chunks)):
            issue_xin(c)

        for c in range(n_chunks):
            p = c % _NRING
            nc = c + 1
            if _NRING - 1 <= nc < n_chunks:
                np_ = nc % _NRING
                if o_cp[np_] is not None:
                    for cp in o_cp[np_]:
                        cp.wait()
                    o_cp[np_] = None
                issue_xin(nc)
            if c + 1 < n_chunks:
                t_cp[(c + 1) % 2] = pltpu.async_copy(
                    tab_hbm.at[pl.ds(row_base + (c + 1) * chunk, chunk)],
                    tbufs[(c + 1) % 2], st)

            t_cp[c % 2].wait()
            t_cp[c % 2] = None
            for cp in x_cp[p]:
                cp.wait()
            x_cp[p] = None

            tb = tbufs[c % 2]
            xbs = xbufs[p]

            def col_body(jj, carry, tb=tb, xbs=xbs):
                jbase = jj * (_UNROLL * _LANES)
                for r in range(chunk):
                    for u in range(_UNROLL):
                        off = jbase + u * _LANES
                        tv = tb[r, pl.ds(off, _LANES)]
                        for b in range(batch):
                            plsc.addupdate(
                                xbs[b].at[r, pl.ds(off, _LANES)], tv)
                return carry

            lax.fori_loop(0, vregs_per_row // _UNROLL, col_body, 0)

            cps = []
            for b in range(batch):
                cps.append(pltpu.async_copy(
                    xbs[b],
                    out_hbm.at[b, pl.ds(row_base + c * chunk, chunk)], so))
            o_cp[p] = cps

        for i in range(_NRING):
            p = (n_chunks + i) % _NRING
            if o_cp[p] is not None:
                for cp in o_cp[p]:
                    cp.wait()
                o_cp[p] = None

    return sc_kernel


def kernel(x, pos_table):
    batch, seq_len, d_model = x.shape
    return _make_sc_kernel(batch, seq_len, d_model)(x, pos_table)
